# baseline (speedup 1.0000x reference)
import jax
import jax.numpy as jnp
from jax.experimental import pallas as pl
from jax.experimental.pallas import tpu as pltpu

_EPS = 1e-06
_BB = 2
_C = 512


def _pcen_kernel(x_ref, a_ref, d_ref, scal_ref, o_ref):
    ac = scal_ref[0]
    dc = scal_ref[1]
    rc = scal_ref[2]
    drc = scal_ref[3]
    a = a_ref[...]
    d = d_ref[...]
    T = x_ref.shape[-1]

    carry = x_ref[:, :, 0:1].reshape(_BB * 128, 1)
    for c in range(T // _C):
        ts = slice(c * _C, (c + 1) * _C)
        xs = x_ref[:, :, ts].reshape(_BB * 128, _C)
        m = jnp.dot(xs, a, preferred_element_type=jnp.float32) + carry * d
        carry = m[:, _C - 1:_C]
        l = jnp.log2(_EPS + m)
        g = jnp.exp2(ac * l)
        pcen = jnp.exp2(rc * (jnp.log2(xs + dc * g) - ac * l)) - drc
        o_ref[:, :, ts] = pcen.reshape(_BB, 128, _C)


def _pcen_call(mel_spec, a_mat, d_vec, scal):
    B, F, T = mel_spec.shape
    return pl.pallas_call(
        _pcen_kernel,
        out_shape=jax.ShapeDtypeStruct((B, F, T), jnp.float32),
        grid=(B // _BB,),
        in_specs=[
            pl.BlockSpec((_BB, F, T), lambda b: (b, 0, 0)),
            pl.BlockSpec((_C, _C), lambda b: (0, 0)),
            pl.BlockSpec((1, _C), lambda b: (0, 0)),
            pl.BlockSpec(memory_space=pltpu.SMEM),
        ],
        out_specs=pl.BlockSpec((_BB, F, T), lambda b: (b, 0, 0)),
        compiler_params=pltpu.CompilerParams(
            dimension_semantics=("parallel",),
            vmem_limit_bytes=56 * 1024 * 1024,
        ),
        name="pcen_fused",
    )(mel_spec, a_mat, d_vec, scal)


@jax.jit
def kernel(mel_spec, alpha, delta, r, s):
    ac = jnp.clip(alpha, 0.01, 0.99)
    dc = jnp.abs(delta) + _EPS
    rc = jnp.clip(r, 0.01, 1.0)
    scal = jnp.stack([ac, dc, rc, dc**rc]).astype(jnp.float32)

    i = jnp.arange(_C, dtype=jnp.float32)[:, None]
    j = jnp.arange(_C, dtype=jnp.float32)[None, :]
    decay = jnp.power(1.0 - s, jnp.maximum(j - i, 0.0))
    a_mat = jnp.where(i <= j, s * decay, 0.0).astype(jnp.float32)
    d_vec = jnp.power(1.0 - s, j + 1.0).astype(jnp.float32)

    return _pcen_call(mel_spec, a_mat, d_vec, scal)


# speedup vs baseline: 1.0155x; 1.0155x over previous
import jax
import jax.numpy as jnp
from jax.experimental import pallas as pl
from jax.experimental.pallas import tpu as pltpu

_EPS = 1e-06
_BB = 32
_C = 512


def _pcen_kernel(xlo_ref, xhi_ref, a_ref, d_ref, scal_ref, o_ref, carry_ref):
    t = pl.program_id(1)

    @pl.when(t == 0)
    def _():
        carry_ref[:, 0:64, :] = xlo_ref[:, :, 0:1]
        carry_ref[:, 64:128, :] = xhi_ref[:, :, 0:1]

    ac = scal_ref[0]
    dc = scal_ref[1]
    rc = scal_ref[2]
    drc = scal_ref[3]
    a = a_ref[...]
    d = d_ref[...]

    for i in range(_BB):
        for h, x_ref in ((0, xlo_ref), (1, xhi_ref)):
            fs = slice(64 * h, 64 * h + 64)
            xs = x_ref[i]
            carry = carry_ref[i, fs]
            m = jnp.dot(xs, a, preferred_element_type=jnp.float32) + carry * d
            carry_ref[i, fs] = m[:, _C - 1:_C]
            l = jnp.log2(_EPS + m)
            g = jnp.exp2(ac * l)
            o_ref[i, fs] = jnp.exp2(rc * (jnp.log2(xs + dc * g) - ac * l)) - drc


def _pcen_call(mel_spec, a_mat, d_vec, scal):
    B, F, T = mel_spec.shape
    grid = (B // _BB, T // _C)
    return pl.pallas_call(
        _pcen_kernel,
        out_shape=jax.ShapeDtypeStruct((B, F, T), jnp.float32),
        grid=grid,
        in_specs=[
            pl.BlockSpec((_BB, 64, _C), lambda b, t: (b, 0, t)),
            pl.BlockSpec((_BB, 64, _C), lambda b, t: (b, 1, t)),
            pl.BlockSpec((_C, _C), lambda b, t: (0, 0)),
            pl.BlockSpec((1, _C), lambda b, t: (0, 0)),
            pl.BlockSpec(memory_space=pltpu.SMEM),
        ],
        out_specs=pl.BlockSpec((_BB, F, _C), lambda b, t: (b, 0, t)),
        scratch_shapes=[pltpu.VMEM((_BB, 128, 1), jnp.float32)],
        compiler_params=pltpu.CompilerParams(
            dimension_semantics=("parallel", "arbitrary"),
            vmem_limit_bytes=56 * 1024 * 1024,
        ),
        name="pcen_fused",
    )(mel_spec, mel_spec, a_mat, d_vec, scal)


@jax.jit
def kernel(mel_spec, alpha, delta, r, s):
    ac = jnp.clip(alpha, 0.01, 0.99)
    dc = jnp.abs(delta) + _EPS
    rc = jnp.clip(r, 0.01, 1.0)
    scal = jnp.stack([ac, dc, rc, dc**rc]).astype(jnp.float32)

    i = jnp.arange(_C, dtype=jnp.float32)[:, None]
    j = jnp.arange(_C, dtype=jnp.float32)[None, :]
    decay = jnp.power(1.0 - s, jnp.maximum(j - i, 0.0))
    a_mat = jnp.where(i <= j, s * decay, 0.0).astype(jnp.float32)
    d_vec = jnp.power(1.0 - s, j + 1.0).astype(jnp.float32)

    return _pcen_call(mel_spec, a_mat, d_vec, scal)


# speedup vs baseline: 1.0472x; 1.0313x over previous
import jax
import jax.numpy as jnp
from jax.experimental import pallas as pl
from jax.experimental.pallas import tpu as pltpu

_EPS = 1e-06
_BB = 32
_C = 512


def _pcen_kernel(x_ref, scal_ref, o_ref, carry_ref, a_ref, d_ref):
    t = pl.program_id(1)
    s = scal_ref[3]
    ac = jnp.clip(scal_ref[0], 0.01, 0.99)
    dc = jnp.abs(scal_ref[1]) + _EPS
    rc = jnp.clip(scal_ref[2], 0.01, 1.0)

    @pl.when((pl.program_id(0) == 0) & (t == 0))
    def _():
        ii = jax.lax.broadcasted_iota(jnp.int32, (_C, _C), 0)
        jj = jax.lax.broadcasted_iota(jnp.int32, (_C, _C), 1)
        lag = (jj - ii).astype(jnp.float32)
        l1ms = jnp.log2(1.0 - s)
        a_ref[...] = jnp.where(ii <= jj, s * jnp.exp2(lag * l1ms), 0.0)
        jrow = jax.lax.broadcasted_iota(jnp.int32, (1, _C), 1).astype(jnp.float32)
        d_ref[...] = jnp.exp2((jrow + 1.0) * l1ms)

    @pl.when(t == 0)
    def _():
        carry_ref[...] = x_ref[:, :, 0:1]

    a = a_ref[...]
    d = d_ref[...]
    drc = jnp.exp2(rc * jnp.log2(jnp.full((1, _C), dc, jnp.float32)))

    for i in range(_BB):
        xs = x_ref[i]
        carry = carry_ref[i]
        m = jnp.dot(xs, a, preferred_element_type=jnp.float32) + carry * d
        carry_ref[i] = m[:, _C - 1:_C]
        l = jnp.log2(_EPS + m)
        g = jnp.exp2(ac * l)
        o_ref[i] = jnp.exp2(rc * (jnp.log2(xs + dc * g) - ac * l)) - drc


@jax.jit
def kernel(mel_spec, alpha, delta, r, s):
    B, F, T = mel_spec.shape
    scal = jnp.stack([alpha, delta, r, s]).astype(jnp.float32)
    grid = (B // _BB, T // _C)
    return pl.pallas_call(
        _pcen_kernel,
        out_shape=jax.ShapeDtypeStruct((B, F, T), jnp.float32),
        grid=grid,
        in_specs=[
            pl.BlockSpec((_BB, F, _C), lambda b, t: (b, 0, t)),
            pl.BlockSpec(memory_space=pltpu.SMEM),
        ],
        out_specs=pl.BlockSpec((_BB, F, _C), lambda b, t: (b, 0, t)),
        scratch_shapes=[
            pltpu.VMEM((_BB, 128, 1), jnp.float32),
            pltpu.VMEM((_C, _C), jnp.float32),
            pltpu.VMEM((1, _C), jnp.float32),
        ],
        compiler_params=pltpu.CompilerParams(
            dimension_semantics=("parallel", "arbitrary"),
            vmem_limit_bytes=56 * 1024 * 1024,
        ),
        name="pcen_fused",
    )(mel_spec, scal)
